# baseline (speedup 1.0000x reference)
import functools

import jax
import jax.numpy as jnp
from jax import lax
from jax.experimental import pallas as pl
from jax.experimental.pallas import tpu as pltpu
from jax.experimental.pallas import tpu_sc as plsc

H = 768
L = 16
NC = 2
NS = 16
NW = NC * NS
HV = H // L
EPS = 1e-5
CHUNK = 32
INV_H = 1.0 / H


def _rsqrt_vec(x):
  i = lax.bitcast_convert_type(x, jnp.int32)
  i = jnp.int32(0x5F3759DF) - lax.shift_right_arithmetic(i, 1)
  y = lax.bitcast_convert_type(i, jnp.float32)
  half_x = x * jnp.float32(0.5)
  for _ in range(2):
    y = y * (jnp.float32(1.5) - half_x * y * y)
  return y


def _ln_chunk(inb, outb, gam, rstd_ref, mb_ref):
  @plsc.parallel_loop(0, CHUNK, unroll=2)
  def stats_body(r):
    s = jnp.zeros((L,), jnp.float32)
    q = jnp.zeros((L,), jnp.float32)
    for j in range(HV):
      v = inb[r, pl.ds(j * L, L)]
      s = s + v
      q = q + v * v
    ssum = jnp.sum(s)
    qsum = jnp.sum(q)
    mean = lax.broadcast_in_dim(ssum, (L,), ()) * jnp.float32(INV_H)
    ex2 = lax.broadcast_in_dim(qsum, (L,), ()) * jnp.float32(INV_H)
    var = ex2 - mean * mean
    rstd = _rsqrt_vec(var + jnp.float32(EPS))
    rstd_ref[r, :] = rstd
    mb_ref[r, :] = mean * rstd

  jw = HV // 4

  def jblock_body(jb, carry):
    base = pl.multiple_of(jb * (jw * L), jw * L)
    gvs = [gam[pl.ds(base + t * L, L)] for t in range(jw)]

    @plsc.parallel_loop(0, CHUNK, unroll=4)
    def apply_body(r):
      a = rstd_ref[r, :]
      mb = mb_ref[r, :]
      for t in range(jw):
        v = inb[r, pl.ds(base + t * L, L)]
        outb[r, pl.ds(base + t * L, L)] = (v * a - mb) * gvs[t]

    return carry

  lax.fori_loop(0, 4, jblock_body, 0)


def _body(n_tok, ids_hbm, table_hbm, out_hbm,
          idx_v, ring, gam, rstd_v, mb_v, gsem, osem):
  rows_per_w = n_tok // NW
  nch = rows_per_w // CHUNK

  wid = lax.axis_index("s") * NC + lax.axis_index("c")
  row_base = pl.multiple_of(wid * rows_per_w, rows_per_w)

  pltpu.sync_copy(ids_hbm.at[pl.ds(row_base, rows_per_w)], idx_v)

  def slot(par):
    return ring.at[pl.ds(pl.multiple_of(par * CHUNK, CHUNK), CHUNK)]

  def do_gather(c, par):
    off = pl.multiple_of(c * CHUNK, CHUNK)
    pltpu.async_copy(table_hbm.at[idx_v.at[pl.ds(off, CHUNK)]], slot(par),
                     gsem.at[par])

  def wait_gather(par):
    pltpu.make_async_copy(table_hbm.at[pl.ds(0, CHUNK)], slot(par),
                          gsem.at[par]).wait()

  def do_out(c, par):
    off = pl.multiple_of(row_base + c * CHUNK, CHUNK)
    pltpu.async_copy(slot(par), out_hbm.at[pl.ds(off, CHUNK)], osem.at[par])

  def wait_out(par):
    pltpu.make_async_copy(slot(par), out_hbm.at[pl.ds(0, CHUNK)],
                          osem.at[par]).wait()

  do_gather(0, 0)
  do_gather(1, 1)

  def gbody(c, carry):
    par = lax.rem(c, 4)
    par2 = lax.rem(c + 2, 4)
    pl.when(c >= 2)(lambda: wait_out(par2))
    pl.when(c < nch - 2)(lambda: do_gather(c + 2, par2))
    wait_gather(par)
    _ln_chunk(slot(par), slot(par), gam, rstd_v, mb_v)
    do_out(c, par)
    return carry

  lax.fori_loop(0, nch, gbody, 0)

  for par in (2, 3):
    wait_out(par)


def _body_with_gamma(n_tok, ids_hbm, table_hbm, gamma_hbm, out_hbm,
                     idx_v, ring, gam, rstd_v, mb_v, gsem, osem):
  pltpu.sync_copy(gamma_hbm, gam)
  _body(n_tok, ids_hbm, table_hbm, out_hbm,
        idx_v, ring, gam, rstd_v, mb_v, gsem, osem)


def kernel(input_ids, table, gamma):
  b, s = input_ids.shape
  n_tok = b * s
  rows_per_w = n_tok // NW
  ids_flat = input_ids.reshape((n_tok,))
  mesh = plsc.VectorSubcoreMesh(core_axis_name="c", subcore_axis_name="s")
  run = pl.kernel(
      functools.partial(_body_with_gamma, n_tok),
      out_type=jax.ShapeDtypeStruct((n_tok, H), jnp.float32),
      mesh=mesh,
      compiler_params=pltpu.CompilerParams(needs_layout_passes=False),
      scratch_types=[
          pltpu.VMEM((rows_per_w,), jnp.int32),
          pltpu.VMEM((4 * CHUNK, H), jnp.float32),
          pltpu.VMEM((H,), jnp.float32),
          pltpu.VMEM((CHUNK, L), jnp.float32),
          pltpu.VMEM((CHUNK, L), jnp.float32),
          pltpu.SemaphoreType.DMA((4,)),
          pltpu.SemaphoreType.DMA((4,)),
      ],
  )
  out = run(ids_flat, table, gamma)
  return out.reshape((b, s, H))


# speedup vs baseline: 1.0106x; 1.0106x over previous
import functools

import jax
import jax.numpy as jnp
from jax import lax
from jax.experimental import pallas as pl
from jax.experimental.pallas import tpu as pltpu
from jax.experimental.pallas import tpu_sc as plsc

H = 768
L = 16
NC = 2
NS = 16
NW = NC * NS
HV = H // L
EPS = 1e-5
CHUNK = 32
INV_H = 1.0 / H


def _rsqrt_vec(x):
  i = lax.bitcast_convert_type(x, jnp.int32)
  i = jnp.int32(0x5F3759DF) - lax.shift_right_arithmetic(i, 1)
  y = lax.bitcast_convert_type(i, jnp.float32)
  half_x = x * jnp.float32(0.5)
  for _ in range(2):
    y = y * (jnp.float32(1.5) - half_x * y * y)
  return y


def _ln_chunk(inb, outb, gam, rstd_ref, mb_ref):
  @plsc.parallel_loop(0, CHUNK, unroll=2)
  def stats_body(r):
    s = jnp.zeros((L,), jnp.float32)
    q = jnp.zeros((L,), jnp.float32)
    for j in range(HV):
      v = inb[r, pl.ds(j * L, L)]
      s = s + v
      q = q + v * v
    ssum = jnp.sum(s)
    qsum = jnp.sum(q)
    mean = lax.broadcast_in_dim(ssum, (L,), ()) * jnp.float32(INV_H)
    ex2 = lax.broadcast_in_dim(qsum, (L,), ()) * jnp.float32(INV_H)
    var = ex2 - mean * mean
    rstd = _rsqrt_vec(var + jnp.float32(EPS))
    rstd_ref[r, :] = rstd
    mb_ref[r, :] = mean * rstd

  jw = HV // 4

  def jblock_body(jb, carry):
    base = pl.multiple_of(jb * (jw * L), jw * L)
    gvs = [gam[pl.ds(base + t * L, L)] for t in range(jw)]

    @plsc.parallel_loop(0, CHUNK, unroll=2)
    def apply_body(r):
      a = rstd_ref[r, :]
      mb = mb_ref[r, :]
      for t in range(jw):
        v = inb[r, pl.ds(base + t * L, L)]
        outb[r, pl.ds(base + t * L, L)] = (v * a - mb) * gvs[t]

    return carry

  lax.fori_loop(0, 4, jblock_body, 0)


def _body(n_tok, ids_hbm, table_hbm, out_hbm,
          idx_v, ring, gam, rstd_v, mb_v, gsem, osem):
  rows_per_w = n_tok // NW
  nch = rows_per_w // CHUNK

  wid = lax.axis_index("s") * NC + lax.axis_index("c")
  row_base = pl.multiple_of(wid * rows_per_w, rows_per_w)

  pltpu.sync_copy(ids_hbm.at[pl.ds(row_base, rows_per_w)], idx_v)

  def slot(par):
    return ring.at[pl.ds(pl.multiple_of(par * CHUNK, CHUNK), CHUNK)]

  def do_gather(c, par):
    off = pl.multiple_of(c * CHUNK, CHUNK)
    pltpu.async_copy(table_hbm.at[idx_v.at[pl.ds(off, CHUNK)]], slot(par),
                     gsem.at[par])

  def wait_gather(par):
    pltpu.make_async_copy(table_hbm.at[pl.ds(0, CHUNK)], slot(par),
                          gsem.at[par]).wait()

  def do_out(c, par):
    off = pl.multiple_of(row_base + c * CHUNK, CHUNK)
    pltpu.async_copy(slot(par), out_hbm.at[pl.ds(off, CHUNK)], osem.at[par])

  def wait_out(par):
    pltpu.make_async_copy(slot(par), out_hbm.at[pl.ds(0, CHUNK)],
                          osem.at[par]).wait()

  do_gather(0, 0)
  do_gather(1, 1)

  def gbody(c, carry):
    par = lax.rem(c, 4)
    par2 = lax.rem(c + 2, 4)
    pl.when(c >= 2)(lambda: wait_out(par2))
    pl.when(c < nch - 2)(lambda: do_gather(c + 2, par2))
    wait_gather(par)
    _ln_chunk(slot(par), slot(par), gam, rstd_v, mb_v)
    do_out(c, par)
    return carry

  lax.fori_loop(0, nch, gbody, 0)

  for par in (2, 3):
    wait_out(par)


def _body_with_gamma(n_tok, ids_hbm, table_hbm, gamma_hbm, out_hbm,
                     idx_v, ring, gam, rstd_v, mb_v, gsem, osem):
  pltpu.sync_copy(gamma_hbm, gam)
  _body(n_tok, ids_hbm, table_hbm, out_hbm,
        idx_v, ring, gam, rstd_v, mb_v, gsem, osem)


def kernel(input_ids, table, gamma):
  b, s = input_ids.shape
  n_tok = b * s
  rows_per_w = n_tok // NW
  ids_flat = input_ids.reshape((n_tok,))
  mesh = plsc.VectorSubcoreMesh(core_axis_name="c", subcore_axis_name="s")
  run = pl.kernel(
      functools.partial(_body_with_gamma, n_tok),
      out_type=jax.ShapeDtypeStruct((n_tok, H), jnp.float32),
      mesh=mesh,
      compiler_params=pltpu.CompilerParams(needs_layout_passes=False),
      scratch_types=[
          pltpu.VMEM((rows_per_w,), jnp.int32),
          pltpu.VMEM((4 * CHUNK, H), jnp.float32),
          pltpu.VMEM((H,), jnp.float32),
          pltpu.VMEM((CHUNK, L), jnp.float32),
          pltpu.VMEM((CHUNK, L), jnp.float32),
          pltpu.SemaphoreType.DMA((4,)),
          pltpu.SemaphoreType.DMA((4,)),
      ],
  )
  out = run(ids_flat, table, gamma)
  return out.reshape((b, s, H))
